# baseline (speedup 1.0000x reference)
import functools

import jax
import jax.numpy as jnp
from jax import lax
from jax.experimental import pallas as pl
from jax.experimental.pallas import tpu as pltpu
from jax.experimental.pallas import tpu_sc as plsc

_N = 320000
_P = 80000
_D = 128
_NEG = -1000000000.0

_NW = 32
_PPW = _P // _NW
_CHUNK = 128
_PB = 500
_NWIN = _PPW // _PB
_KCH = 2000
_RPW = _N // _NW
_L = 16
_NST = _NW * 8 + 16


def _sc_pool(features_flat, keys_pad, kprev_pad, starts_p):
    mesh = plsc.VectorSubcoreMesh(core_axis_name="c", subcore_axis_name="s")

    @functools.partial(
        pl.kernel,
        out_type=(
            jax.ShapeDtypeStruct((_P * _D,), jnp.float32),
            jax.ShapeDtypeStruct((_N,), jnp.int32),
        ),
        mesh=mesh,
        compiler_params=pltpu.CompilerParams(
            use_tc_tiling_on_sc=False, needs_layout_passes=False),
        scratch_types=[
            pltpu.VMEM((2 * _CHUNK * _D,), jnp.float32),
            pltpu.VMEM((2 * _CHUNK,), jnp.int32),
            pltpu.VMEM((2 * _CHUNK,), jnp.int32),
            pltpu.VMEM((_PB * _D,), jnp.float32),
            pltpu.VMEM((_NST,), jnp.int32),
            pltpu.VMEM((_KCH,), jnp.int32),
            pltpu.SemaphoreType.DMA,
            pltpu.SemaphoreType.DMA,
        ],
    )
    def body(feat_hbm, keys_hbm, kprev_hbm, starts_hbm, pooled_hbm, pidx_hbm,
             feat_v, keys_v, kprev_v, stage_v, starts_v, kio_v, sem0, sem1):
        sems = (sem0, sem1)
        wid = lax.axis_index("c") * 16 + lax.axis_index("s")
        wp0 = wid * _PPW
        zero = jnp.zeros((_L,), jnp.float32)
        negv = jnp.full((_L,), _NEG, jnp.float32)

        pltpu.sync_copy(starts_hbm, starts_v)
        sv = starts_v[pl.ds(wid * 8, _L)]
        rs = sv[0]
        re = sv[1]
        rs128 = (rs // _CHUNK) * _CHUNK
        nchunks = (re - rs128 + _CHUNK - 1) // _CHUNK

        def start_chunk(c, b):
            row0 = rs128 + c * _CHUNK
            frow0 = jnp.minimum(row0, _N - _CHUNK)
            pltpu.async_copy(
                feat_hbm.at[pl.ds(frow0 * _D, _CHUNK * _D)],
                feat_v.at[pl.ds(b * _CHUNK * _D, _CHUNK * _D)], sems[b])
            pltpu.async_copy(
                keys_hbm.at[pl.ds(row0, _CHUNK)],
                keys_v.at[pl.ds(b * _CHUNK, _CHUNK)], sems[b])
            pltpu.async_copy(
                kprev_hbm.at[pl.ds(row0, _CHUNK)],
                kprev_v.at[pl.ds(b * _CHUNK, _CHUNK)], sems[b])

        def wait_chunk(b):
            pltpu.make_async_copy(
                feat_hbm.at[pl.ds(0, _CHUNK * _D)],
                feat_v.at[pl.ds(b * _CHUNK * _D, _CHUNK * _D)], sems[b]).wait()
            pltpu.make_async_copy(
                keys_hbm.at[pl.ds(0, _CHUNK)],
                keys_v.at[pl.ds(b * _CHUNK, _CHUNK)], sems[b]).wait()
            pltpu.make_async_copy(
                keys_hbm.at[pl.ds(0, _CHUNK)],
                kprev_v.at[pl.ds(b * _CHUNK, _CHUNK)], sems[b]).wait()

        start_chunk(jnp.int32(0), 0)
        start_chunk(jnp.int32(1), 1)

        for c in range(_RPW // _KCH):
            base = wid * _RPW + c * _KCH
            pltpu.sync_copy(keys_hbm.at[pl.ds(base, _KCH)], kio_v)

            def shift_body(v, carry):
                x = kio_v[pl.ds(v * _L, _L)]
                kio_v[pl.ds(v * _L, _L)] = x >> 2
                return carry

            lax.fori_loop(jnp.int32(0), jnp.int32(_KCH // _L), shift_body, 0)
            pltpu.sync_copy(kio_v, pidx_hbm.at[pl.ds(base, _KCH)])

        def memset_stage():
            def mrow(r, carry):
                for j in range(_D // _L):
                    stage_v[pl.ds(r * _D + j * _L, _L)] = zero
                return carry

            lax.fori_loop(jnp.int32(0), jnp.int32(_PB), mrow, 0)

        memset_stage()

        def flush_win(k):
            pltpu.sync_copy(
                stage_v, pooled_hbm.at[pl.ds((wp0 + k * _PB) * _D, _PB * _D)])
            memset_stage()

        def emit(cur_p, nwf, accs):
            off = (cur_p - wp0 - nwf * _PB) * _D
            for j in range(_D // _L):
                v = jnp.maximum(accs[j], negv)
                v = jnp.where(v == negv, zero, v)
                stage_v[pl.ds(off + j * _L, _L)] = v

        pw2 = jnp.left_shift(jnp.int32(1), lax.iota(jnp.int32, _L))
        zi = jnp.zeros((_L,), jnp.int32)

        def row_body(b):
            def rb(g, carry):
                kv = keys_v[pl.ds(b * _CHUNK + g * _L, _L)] >> 2
                kpv = kprev_v[pl.ds(b * _CHUNK + g * _L, _L)] >> 2
                val_v = (kv >= wp0) & (kv < wp0 + _PPW)
                start_v = val_v & (kv != kpv)
                m_chg = jnp.sum(jnp.where(start_v, pw2, zi), dtype=jnp.int32)
                m_val = jnp.sum(jnp.where(val_v, pw2, zi), dtype=jnp.int32)
                for jj in range(_L):
                    last_p, nwf = carry[0], carry[1]
                    accs = carry[2:]
                    foff = b * _CHUNK * _D + (g * _L + jj) * _D
                    changed = (m_chg & (1 << jj)) != 0
                    validb = (m_val & (1 << jj)) != 0

                    def slow(last_p=last_p, nwf=nwf, accs=accs):
                        p = kv[jj]

                        @pl.when(last_p >= 0)
                        def _():
                            emit(last_p, nwf, accs)

                        wp = (p - wp0) // _PB
                        do_flush = wp > nwf

                        @pl.when(do_flush)
                        def _():
                            def fw(k, carry2):
                                flush_win(k)
                                return carry2

                            lax.fori_loop(nwf, wp, fw, 0)

                        return (jnp.where(do_flush, wp, nwf), p)

                    nwf2, last_p2 = lax.cond(
                        changed, slow, lambda nwf=nwf, last_p=last_p: (nwf, last_p))
                    new_accs = []
                    for j in range(_D // _L):
                        f = feat_v[pl.ds(foff + j * _L, _L)]
                        m = jnp.maximum(accs[j], f)
                        new_accs.append(
                            jnp.where(changed, f, jnp.where(validb, m, accs[j])))
                    carry = (last_p2, nwf2, *new_accs)
                return carry

            return rb

        carry0 = (jnp.int32(-1), jnp.int32(0)) + tuple(zero for _ in range(_D // _L))

        def outer(c2, carry):
            for b in range(2):
                c = 2 * c2 + b
                wait_chunk(b)
                carry = lax.fori_loop(
                    jnp.int32(0), jnp.int32(_CHUNK // _L), row_body(b), carry)
                start_chunk(c + 2, b)
            return carry

        carry = lax.fori_loop(jnp.int32(0), (nchunks + 1) // 2, outer, carry0)
        wait_chunk(0)
        wait_chunk(1)

        cur_p, nwf = carry[0], carry[1]
        accs = carry[2:]

        @pl.when(cur_p >= 0)
        def _():
            emit(cur_p, nwf, accs)

        def fw(k, carry2):
            flush_win(k)
            return carry2

        lax.fori_loop(nwf, jnp.int32(_NWIN), fw, 0)

    return body(features_flat, keys_pad, kprev_pad, starts_p)


_PAD = 512
_SENT = 1 << 29


def kernel(features, keys, parent_level_keys):
    keys32 = keys.astype(jnp.int32)
    keys_pad = jnp.concatenate([keys32, jnp.full((_PAD,), _SENT, jnp.int32)])
    kprev_pad = jnp.concatenate(
        [jnp.full((1,), _SENT, jnp.int32), keys32[:-1],
         jnp.full((_PAD,), _SENT, jnp.int32)])
    bounds = jnp.arange(_NW + 1).astype(keys.dtype) * (4 * _PPW)
    starts = jnp.searchsorted(keys, bounds).astype(jnp.int32)
    starts_p = (
        jnp.zeros((_NST,), jnp.int32)
        .at[8 * jnp.arange(_NW + 1)].set(starts)
        .at[8 * jnp.arange(_NW) + 1].set(starts[1:])
    )
    pooled_flat, pidx = _sc_pool(features.reshape(-1), keys_pad, kprev_pad, starts_p)
    return (pooled_flat.reshape(_P, _D), pidx)


# speedup vs baseline: 1.0632x; 1.0632x over previous
import functools

import jax
import jax.numpy as jnp
from jax import lax
from jax.experimental import pallas as pl
from jax.experimental.pallas import tpu as pltpu
from jax.experimental.pallas import tpu_sc as plsc

_N = 320000
_P = 80000
_D = 128
_NEG = -1000000000.0

_NW = 32
_PPW = _P // _NW
_CHUNK = 128
_PB = 500
_NWIN = _PPW // _PB
_KCH = 2000
_RPW = _N // _NW
_L = 16
_NST = _NW * 8 + 16


def _sc_pool(features_flat, keys_pad, kprev_pad, starts_p):
    mesh = plsc.VectorSubcoreMesh(core_axis_name="c", subcore_axis_name="s")

    @functools.partial(
        pl.kernel,
        out_type=(
            jax.ShapeDtypeStruct((_P * _D,), jnp.float32),
            jax.ShapeDtypeStruct((_N,), jnp.int32),
        ),
        mesh=mesh,
        compiler_params=pltpu.CompilerParams(
            use_tc_tiling_on_sc=False, needs_layout_passes=False),
        scratch_types=[
            pltpu.VMEM((2 * _CHUNK * _D,), jnp.float32),
            pltpu.VMEM((2 * _CHUNK,), jnp.int32),
            pltpu.VMEM((2 * _CHUNK,), jnp.int32),
            pltpu.VMEM((_PB * _D,), jnp.float32),
            pltpu.VMEM((_NST,), jnp.int32),
            pltpu.VMEM((_KCH,), jnp.int32),
            pltpu.SemaphoreType.DMA,
            pltpu.SemaphoreType.DMA,
        ],
    )
    def body(feat_hbm, keys_hbm, kprev_hbm, starts_hbm, pooled_hbm, pidx_hbm,
             feat_v, keys_v, kprev_v, stage_v, starts_v, kio_v, sem0, sem1):
        sems = (sem0, sem1)
        wid = lax.axis_index("c") * 16 + lax.axis_index("s")
        wp0 = wid * _PPW
        zero = jnp.zeros((_L,), jnp.float32)
        negv = jnp.full((_L,), _NEG, jnp.float32)

        pltpu.sync_copy(starts_hbm, starts_v)
        sv = starts_v[pl.ds(wid * 8, _L)]
        rs = sv[0]
        re = sv[1]
        rs128 = (rs // _CHUNK) * _CHUNK
        nchunks = (re - rs128 + _CHUNK - 1) // _CHUNK

        def start_chunk(c, b):
            row0 = rs128 + c * _CHUNK
            frow0 = jnp.minimum(row0, _N - _CHUNK)
            pltpu.async_copy(
                feat_hbm.at[pl.ds(frow0 * _D, _CHUNK * _D)],
                feat_v.at[pl.ds(b * _CHUNK * _D, _CHUNK * _D)], sems[b])
            pltpu.async_copy(
                keys_hbm.at[pl.ds(row0, _CHUNK)],
                keys_v.at[pl.ds(b * _CHUNK, _CHUNK)], sems[b])
            pltpu.async_copy(
                kprev_hbm.at[pl.ds(row0, _CHUNK)],
                kprev_v.at[pl.ds(b * _CHUNK, _CHUNK)], sems[b])

        def wait_chunk(b):
            pltpu.make_async_copy(
                feat_hbm.at[pl.ds(0, _CHUNK * _D)],
                feat_v.at[pl.ds(b * _CHUNK * _D, _CHUNK * _D)], sems[b]).wait()
            pltpu.make_async_copy(
                keys_hbm.at[pl.ds(0, _CHUNK)],
                keys_v.at[pl.ds(b * _CHUNK, _CHUNK)], sems[b]).wait()
            pltpu.make_async_copy(
                keys_hbm.at[pl.ds(0, _CHUNK)],
                kprev_v.at[pl.ds(b * _CHUNK, _CHUNK)], sems[b]).wait()

        start_chunk(jnp.int32(0), 0)
        start_chunk(jnp.int32(1), 1)

        for c in range(_RPW // _KCH):
            base = wid * _RPW + c * _KCH
            pltpu.sync_copy(keys_hbm.at[pl.ds(base, _KCH)], kio_v)

            def shift_body(v, carry):
                x = kio_v[pl.ds(v * _L, _L)]
                kio_v[pl.ds(v * _L, _L)] = x >> 2
                return carry

            lax.fori_loop(jnp.int32(0), jnp.int32(_KCH // _L), shift_body, 0)
            pltpu.sync_copy(kio_v, pidx_hbm.at[pl.ds(base, _KCH)])

        def memset_stage():
            def mrow(r, carry):
                for j in range(_D // _L):
                    stage_v[pl.ds(r * _D + j * _L, _L)] = zero
                return carry

            lax.fori_loop(jnp.int32(0), jnp.int32(_PB), mrow, 0)

        memset_stage()

        def flush_win(k):
            pltpu.sync_copy(
                stage_v, pooled_hbm.at[pl.ds((wp0 + k * _PB) * _D, _PB * _D)])
            memset_stage()

        def emit(cur_p, nwf, accs):
            off = (cur_p - wp0 - nwf * _PB) * _D
            for j in range(_D // _L):
                v = jnp.maximum(accs[j], negv)
                v = jnp.where(v == negv, zero, v)
                stage_v[pl.ds(off + j * _L, _L)] = v

        pw2 = jnp.left_shift(jnp.int32(1), lax.iota(jnp.int32, _L))
        zi = jnp.zeros((_L,), jnp.int32)

        def row_body(b):
            def rb(g, carry):
                kv = keys_v[pl.ds(b * _CHUNK + g * _L, _L)] >> 2
                kpv = kprev_v[pl.ds(b * _CHUNK + g * _L, _L)] >> 2
                m_chg = jnp.sum(jnp.where(kv != kpv, pw2, zi), dtype=jnp.int32)
                for jj in range(_L):
                    last_p, nwf = carry[0], carry[1]
                    accs = carry[2:]
                    foff = b * _CHUNK * _D + (g * _L + jj) * _D
                    changed = (m_chg & (1 << jj)) != 0

                    def slow(last_p=last_p, nwf=nwf, accs=accs):
                        p = kv[jj]

                        @pl.when((last_p >= wp0) & (last_p < wp0 + _PPW))
                        def _():
                            emit(last_p, nwf, accs)

                        wp = jnp.minimum((p - wp0) // _PB, _NWIN)
                        do_flush = wp > nwf

                        @pl.when(do_flush)
                        def _():
                            def fw(k, carry2):
                                flush_win(k)
                                return carry2

                            lax.fori_loop(nwf, wp, fw, 0)

                        return (jnp.where(do_flush, wp, nwf), p)

                    nwf2, last_p2 = lax.cond(
                        changed, slow, lambda nwf=nwf, last_p=last_p: (nwf, last_p))
                    new_accs = []
                    for j in range(_D // _L):
                        f = feat_v[pl.ds(foff + j * _L, _L)]
                        m = jnp.maximum(accs[j], f)
                        new_accs.append(jnp.where(changed, f, m))
                    carry = (last_p2, nwf2, *new_accs)
                return carry

            return rb

        carry0 = (jnp.int32(-1), jnp.int32(0)) + tuple(zero for _ in range(_D // _L))

        def outer(c2, carry):
            for b in range(2):
                c = 2 * c2 + b
                wait_chunk(b)
                carry = lax.fori_loop(
                    jnp.int32(0), jnp.int32(_CHUNK // _L), row_body(b), carry)
                start_chunk(c + 2, b)
            return carry

        carry = lax.fori_loop(jnp.int32(0), (nchunks + 1) // 2, outer, carry0)
        wait_chunk(0)
        wait_chunk(1)

        cur_p, nwf = carry[0], carry[1]
        accs = carry[2:]

        @pl.when((cur_p >= wp0) & (cur_p < wp0 + _PPW))
        def _():
            emit(cur_p, nwf, accs)

        def fw(k, carry2):
            flush_win(k)
            return carry2

        lax.fori_loop(nwf, jnp.int32(_NWIN), fw, 0)

    return body(features_flat, keys_pad, kprev_pad, starts_p)


_PAD = 512
_SENT = 1 << 29


def kernel(features, keys, parent_level_keys):
    keys32 = keys.astype(jnp.int32)
    keys_pad = jnp.concatenate([keys32, jnp.full((_PAD,), _SENT, jnp.int32)])
    kprev_pad = jnp.concatenate(
        [jnp.full((1,), _SENT, jnp.int32), keys32[:-1],
         jnp.full((_PAD,), _SENT, jnp.int32)])
    bounds = jnp.arange(_NW + 1).astype(keys.dtype) * (4 * _PPW)
    starts = jnp.searchsorted(keys, bounds).astype(jnp.int32)
    starts_p = (
        jnp.zeros((_NST,), jnp.int32)
        .at[8 * jnp.arange(_NW + 1)].set(starts)
        .at[8 * jnp.arange(_NW) + 1].set(starts[1:])
    )
    pooled_flat, pidx = _sc_pool(features.reshape(-1), keys_pad, kprev_pad, starts_p)
    return (pooled_flat.reshape(_P, _D), pidx)
